# baseline (speedup 1.0000x reference)
import functools

import jax
import jax.numpy as jnp
from jax import lax
from jax.experimental import pallas as pl
from jax.experimental.pallas import tpu as pltpu
from jax.experimental.pallas import tpu_sc as plsc

MAXD = 10
V = MAXD + 1
D = 64
N_SIDE = 1024
B = N_SIDE * N_SIDE
B2 = B // 2
NC = 2
NS = 16
NW = NC * NS
K = 128
NKC = B2 // (NW * K)
L = 16


def _body(eo_hbm, table2_hbm, out_hbm, eo_v, cidx_v, rows_v, sem):
    wid = lax.axis_index("s") * NC + lax.axis_index("c")
    row0 = wid * NKC

    pltpu.sync_copy(eo_hbm.at[:, pl.ds(row0, NKC)], eo_v)

    def combine_row(j, carry):
        for t in range(K // L):
            sl = pl.ds(t * L, L)
            a = eo_v[0, j, sl]
            b = eo_v[1, j, sl]
            a = jnp.minimum(jnp.maximum(a, 0), MAXD)
            b = jnp.minimum(jnp.maximum(b, 0), MAXD)
            cidx_v[j, sl] = a * V + b
        return carry

    lax.fori_loop(0, NKC, combine_row, 0)

    def gather_chunk(j, carry):
        pltpu.async_copy(table2_hbm.at[cidx_v.at[j]], rows_v, sem).wait()
        pltpu.sync_copy(rows_v, out_hbm.at[pl.ds((row0 + j) * K, K)])
        return carry

    lax.fori_loop(0, NKC, gather_chunk, 0)


_gather_call = functools.partial(
    pl.kernel,
    out_type=jax.ShapeDtypeStruct((B2, 2 * D), jnp.float32),
    mesh=plsc.VectorSubcoreMesh(
        core_axis_name="c", subcore_axis_name="s", num_cores=NC, num_subcores=NS
    ),
    scratch_types=[
        pltpu.VMEM((2, NKC, K), jnp.int32),
        pltpu.VMEM((NKC, K), jnp.int32),
        pltpu.VMEM((K, 2 * D), jnp.float32),
        pltpu.SemaphoreType.DMA,
    ],
)(_body)


def kernel(distance_matrix, table):
    eo = distance_matrix.reshape(B2, 2).T.reshape(2, B2 // K, K)
    table2 = jnp.concatenate(
        [jnp.repeat(table, V, axis=0), jnp.tile(table, (V, 1))], axis=1
    )
    out = _gather_call(eo, table2)
    return out.reshape(N_SIDE, N_SIDE, D)


# speedup vs baseline: 1.0035x; 1.0035x over previous
import functools

import jax
import jax.numpy as jnp
from jax import lax
from jax.experimental import pallas as pl
from jax.experimental.pallas import tpu as pltpu
from jax.experimental.pallas import tpu_sc as plsc

MAXD = 10
V = MAXD + 1
D = 64
N_SIDE = 1024
B = N_SIDE * N_SIDE
B2 = B // 2
NC = 2
NS = 16
NW = NC * NS
K = 128
NKC = B2 // (NW * K)
L = 16
NBUF = 4


def _body(eo_hbm, table2_hbm, out_hbm, eo_v, cidx_v, *bufs_and_sems):
    rows = bufs_and_sems[:NBUF]
    sg = bufs_and_sems[NBUF : 2 * NBUF]
    so = bufs_and_sems[2 * NBUF : 3 * NBUF]

    wid = lax.axis_index("s") * NC + lax.axis_index("c")
    row0 = wid * NKC

    pltpu.sync_copy(eo_hbm.at[:, pl.ds(row0, NKC)], eo_v)

    def combine_row(j, carry):
        for t in range(K // L):
            sl = pl.ds(t * L, L)
            a = eo_v[0, j, sl]
            b = eo_v[1, j, sl]
            a = jnp.minimum(jnp.maximum(a, 0), MAXD)
            b = jnp.minimum(jnp.maximum(b, 0), MAXD)
            cidx_v[j, sl] = a * V + b
        return carry

    lax.fori_loop(0, NKC, combine_row, 0)

    def fire_gather(j, b):
        pltpu.async_copy(table2_hbm.at[cidx_v.at[j]], rows[b], sg[b])

    def fire_write(j, b):
        pltpu.async_copy(rows[b], out_hbm.at[pl.ds((row0 + j) * K, K)], so[b])

    def wait_gather(j, b):
        pltpu.make_async_copy(table2_hbm.at[cidx_v.at[j]], rows[b], sg[b]).wait()

    def wait_write(j, b):
        pltpu.make_async_copy(
            rows[b], out_hbm.at[pl.ds((row0 + j) * K, K)], so[b]
        ).wait()

    for b in range(NBUF):
        fire_gather(b, b)

    def outer(gi, carry):
        g = gi * NBUF
        for b in range(NBUF):
            j = g + b
            wait_gather(j, b)
            fire_write(j, b)
            wait_write(j, b)

            @pl.when(j + NBUF < NKC)
            def _():
                fire_gather(j + NBUF, b)

        return carry

    lax.fori_loop(0, NKC // NBUF, outer, 0)


_gather_call = functools.partial(
    pl.kernel,
    out_type=jax.ShapeDtypeStruct((B2, 2 * D), jnp.float32),
    mesh=plsc.VectorSubcoreMesh(
        core_axis_name="c", subcore_axis_name="s", num_cores=NC, num_subcores=NS
    ),
    scratch_types=(
        [
            pltpu.VMEM((2, NKC, K), jnp.int32),
            pltpu.VMEM((NKC, K), jnp.int32),
        ]
        + [pltpu.VMEM((K, 2 * D), jnp.float32)] * NBUF
        + [pltpu.SemaphoreType.DMA] * (2 * NBUF)
    ),
)(_body)


def kernel(distance_matrix, table):
    eo = distance_matrix.reshape(B2, 2).T.reshape(2, B2 // K, K)
    table2 = jnp.concatenate(
        [jnp.repeat(table, V, axis=0), jnp.tile(table, (V, 1))], axis=1
    )
    out = _gather_call(eo, table2)
    return out.reshape(N_SIDE, N_SIDE, D)


# speedup vs baseline: 5.7467x; 5.7269x over previous
import functools

import jax
import jax.numpy as jnp
from jax import lax
from jax.experimental import pallas as pl
from jax.experimental.pallas import tpu as pltpu
from jax.experimental.pallas import tpu_sc as plsc

MAXD = 10
V = MAXD + 1
D = 64
N_SIDE = 1024
B = N_SIDE * N_SIDE
B2 = B // 2
NC = 2
NS = 16
NW = NC * NS
K = 128
NKC = B2 // (NW * K)
L = 16
NBUF = 4


def _body(eo_hbm, table2_hbm, out_hbm, eo_v, cidx_v, table2_sh, *bufs_and_sems):
    rows = bufs_and_sems[:NBUF]
    sg = bufs_and_sems[NBUF : 2 * NBUF]
    so = bufs_and_sems[2 * NBUF : 3 * NBUF]

    sid = lax.axis_index("s")
    wid = sid * NC + lax.axis_index("c")
    row0 = wid * NKC

    @pl.when(sid == 0)
    def _():
        pltpu.sync_copy(table2_hbm, table2_sh)

    pltpu.sync_copy(eo_hbm.at[:, pl.ds(row0, NKC)], eo_v)

    def combine_row(j, carry):
        for t in range(K // L):
            sl = pl.ds(t * L, L)
            a = eo_v[0, j, sl]
            b = eo_v[1, j, sl]
            a = jnp.minimum(jnp.maximum(a, 0), MAXD)
            b = jnp.minimum(jnp.maximum(b, 0), MAXD)
            cidx_v[j, sl] = a * V + b
        return carry

    lax.fori_loop(0, NKC, combine_row, 0)

    plsc.subcore_barrier()

    def fire_gather(j, b):
        pltpu.async_copy(table2_sh.at[cidx_v.at[j]], rows[b], sg[b])

    def fire_write(j, b):
        pltpu.async_copy(rows[b], out_hbm.at[pl.ds((row0 + j) * K, K)], so[b])

    def wait_gather(j, b):
        pltpu.make_async_copy(table2_sh.at[cidx_v.at[j]], rows[b], sg[b]).wait()

    def wait_write(j, b):
        pltpu.make_async_copy(
            rows[b], out_hbm.at[pl.ds((row0 + j) * K, K)], so[b]
        ).wait()

    for b in range(NBUF):
        fire_gather(b, b)

    def outer(gi, carry):
        g = gi * NBUF
        for b in range(NBUF):
            j = g + b
            wait_gather(j, b)
            fire_write(j, b)
            wait_write(j, b)

            @pl.when(j + NBUF < NKC)
            def _():
                fire_gather(j + NBUF, b)

        return carry

    lax.fori_loop(0, NKC // NBUF, outer, 0)


_gather_call = functools.partial(
    pl.kernel,
    out_type=jax.ShapeDtypeStruct((B2, 2 * D), jnp.float32),
    mesh=plsc.VectorSubcoreMesh(
        core_axis_name="c", subcore_axis_name="s", num_cores=NC, num_subcores=NS
    ),
    scratch_types=(
        [
            pltpu.VMEM((2, NKC, K), jnp.int32),
            pltpu.VMEM((NKC, K), jnp.int32),
            pltpu.VMEM_SHARED((V * V, 2 * D), jnp.float32),
        ]
        + [pltpu.VMEM((K, 2 * D), jnp.float32)] * NBUF
        + [pltpu.SemaphoreType.DMA] * (2 * NBUF)
    ),
)(_body)


def kernel(distance_matrix, table):
    eo = distance_matrix.reshape(B2, 2).T.reshape(2, B2 // K, K)
    table2 = jnp.concatenate(
        [jnp.repeat(table, V, axis=0), jnp.tile(table, (V, 1))], axis=1
    )
    out = _gather_call(eo, table2)
    return out.reshape(N_SIDE, N_SIDE, D)


# speedup vs baseline: 8.8848x; 1.5461x over previous
import functools

import jax
import jax.numpy as jnp
from jax import lax
from jax.experimental import pallas as pl
from jax.experimental.pallas import tpu as pltpu
from jax.experimental.pallas import tpu_sc as plsc

MAXD = 10
V = MAXD + 1
D = 64
N_SIDE = 1024
B = N_SIDE * N_SIDE
B2 = B // 2
NC = 2
NS = 16
NW = NC * NS
K = 128
NKC = B2 // (NW * K)
NKR = 2 * NKC
L = 16
NBUF = 4


def _body(raw_hbm, table2_hbm, out_hbm, raw_v, cidx_v, table2_sh, *bufs_and_sems):
    rows = bufs_and_sems[:NBUF]
    sg = bufs_and_sems[NBUF : 2 * NBUF]
    so = bufs_and_sems[2 * NBUF : 3 * NBUF]

    sid = lax.axis_index("s")
    wid = sid * NC + lax.axis_index("c")
    row0 = wid * NKC

    @pl.when(sid == 0)
    def _():
        pltpu.sync_copy(table2_hbm, table2_sh)

    pltpu.sync_copy(raw_hbm.at[pl.ds(wid * NKR * K, NKR * K)], raw_v)

    lanes = lax.iota(jnp.int32, L)

    def combine_row(j, carry):
        for t in range(K // L):
            p = j * K + t * L + lanes
            off_e = p * 2
            a = plsc.load_gather(raw_v, [off_e])
            b = plsc.load_gather(raw_v, [off_e + 1])
            a = jnp.minimum(jnp.maximum(a, 0), MAXD)
            b = jnp.minimum(jnp.maximum(b, 0), MAXD)
            cidx_v[j, pl.ds(t * L, L)] = a * V + b
        return carry

    lax.fori_loop(0, NKC, combine_row, 0)

    plsc.subcore_barrier()

    def fire_gather(j, b):
        pltpu.async_copy(table2_sh.at[cidx_v.at[j]], rows[b], sg[b])

    def fire_write(j, b):
        pltpu.async_copy(rows[b], out_hbm.at[pl.ds((row0 + j) * K, K)], so[b])

    def wait_gather(j, b):
        pltpu.make_async_copy(table2_sh.at[cidx_v.at[j]], rows[b], sg[b]).wait()

    def wait_write(j, b):
        pltpu.make_async_copy(
            rows[b], out_hbm.at[pl.ds((row0 + j) * K, K)], so[b]
        ).wait()

    for b in range(NBUF):
        fire_gather(b, b)

    def outer(gi, carry):
        g = gi * NBUF
        for b in range(NBUF):
            j = g + b
            wait_gather(j, b)
            fire_write(j, b)
            wait_write(j, b)

            @pl.when(j + NBUF < NKC)
            def _():
                fire_gather(j + NBUF, b)

        return carry

    lax.fori_loop(0, NKC // NBUF, outer, 0)


_gather_call = functools.partial(
    pl.kernel,
    out_type=jax.ShapeDtypeStruct((B2, 2 * D), jnp.float32),
    mesh=plsc.VectorSubcoreMesh(
        core_axis_name="c", subcore_axis_name="s", num_cores=NC, num_subcores=NS
    ),
    compiler_params=pltpu.CompilerParams(needs_layout_passes=False),
    scratch_types=(
        [
            pltpu.VMEM((NKR * K,), jnp.int32),
            pltpu.VMEM((NKC, K), jnp.int32),
            pltpu.VMEM_SHARED((V * V, 2 * D), jnp.float32),
        ]
        + [pltpu.VMEM((K, 2 * D), jnp.float32)] * NBUF
        + [pltpu.SemaphoreType.DMA] * (2 * NBUF)
    ),
)(_body)


def kernel(distance_matrix, table):
    raw = distance_matrix.reshape(B)
    table2 = jnp.concatenate(
        [jnp.repeat(table, V, axis=0), jnp.tile(table, (V, 1))], axis=1
    )
    out = _gather_call(raw, table2)
    return out.reshape(N_SIDE, N_SIDE, D)
